# Initial kernel scaffold; baseline (speedup 1.0000x reference)
import jax
import jax.numpy as jnp
from jax.experimental import pallas as pl


def kernel(preds, label_threshold_map, label_threshold_mask, label_shrink_map, label_shrink_mask):
    raise NotImplementedError("write your pallas kernel here")


# speedup vs baseline: 17.2017x; 17.2017x over previous
import jax
import jax.numpy as jnp
from jax import lax
from jax.experimental import pallas as pl
from jax.experimental.pallas import tpu as pltpu
from jax.experimental.pallas import tpu_sc as plsc

EPS = 1e-06
ALPHA = 5.0
BETA = 10.0
OHEM_RATIO = 3.0

NC, NS, LANES = 2, 16, 16
NW = NC * NS
CHUNK = 8192
UNROLL = 8
NBINS = 65536


def _dense_body(pr_ref, tm_ref, tmask_ref, g_ref, m_ref, out_ref):
    p0 = pr_ref[0]
    p1 = pr_ref[1]
    p2 = pr_ref[2]
    tm = tm_ref[0]
    tmask = tmask_ref[0]
    g = g_ref[0]
    m = m_ref[0]
    gm = g * m
    p0m = p0 * m
    p2m = p2 * m
    vals = [
        jnp.sum(p0m * g),
        jnp.sum(p0m),
        jnp.sum(gm),
        jnp.sum(m),
        jnp.sum(jnp.abs(p1 - tm) * tmask),
        jnp.sum(tmask),
        jnp.sum(p2m * g),
        jnp.sum(p2m),
    ]
    rows = lax.broadcasted_iota(jnp.int32, (8, 128), 0)
    total = jnp.zeros((8, 128), jnp.float32)
    for i, v in enumerate(vals):
        total = total + jnp.where(rows == i, v, 0.0)

    @pl.when(pl.program_id(0) == 0)
    def _():
        out_ref[...] = jnp.zeros_like(out_ref)

    out_ref[...] += total


def _dense_sums(preds, tm, tmask, g, m):
    B, C, H, W = preds.shape
    pr = preds.reshape(B * C, H, W)
    out = pl.pallas_call(
        _dense_body,
        grid=(B,),
        in_specs=[
            pl.BlockSpec((C, H, W), lambda i: (i, 0, 0)),
            pl.BlockSpec((1, H, W), lambda i: (i, 0, 0)),
            pl.BlockSpec((1, H, W), lambda i: (i, 0, 0)),
            pl.BlockSpec((1, H, W), lambda i: (i, 0, 0)),
            pl.BlockSpec((1, H, W), lambda i: (i, 0, 0)),
        ],
        out_specs=pl.BlockSpec((8, 128), lambda i: (0, 0)),
        out_shape=jax.ShapeDtypeStruct((8, 128), jnp.float32),
    )(pr, tm, tmask, g, m)
    return out[:, 0]


import functools


@functools.lru_cache(maxsize=None)
def _sc_mesh():
    return plsc.VectorSubcoreMesh(core_axis_name="c", subcore_axis_name="s",
                                  num_cores=NC, num_subcores=NS)


def _worker_id():
    return lax.axis_index("s") * NC + lax.axis_index("c")


def _zero_hist(hist):
    zeros = jnp.zeros((LANES,), jnp.int32)

    def zbody(i, _):
        for u in range(UNROLL):
            hist[pl.ds((i * UNROLL + u) * LANES, LANES)] = zeros
        return 0

    lax.fori_loop(0, NBINS // LANES // UNROLL, zbody, 0)


def _neg_bits(g_buf, m_buf, off):
    gv = g_buf[pl.ds(off, LANES)]
    mv = m_buf[pl.ds(off, LANES)]
    neg = mv - gv * mv
    return neg, plsc.bitcast(neg, jnp.uint32)


def _sc_hist_hi_body(g_hbm, m_hbm, out_hbm, g_buf, m_buf, hist):
    wid = _worker_id()
    per_w = g_hbm.shape[0] // NW
    nchunk = per_w // CHUNK
    _zero_hist(hist)
    ones = jnp.ones((LANES,), jnp.int32)
    base = wid * per_w
    for c in range(nchunk):
        pltpu.sync_copy(g_hbm.at[pl.ds(base + c * CHUNK, CHUNK)], g_buf)
        pltpu.sync_copy(m_hbm.at[pl.ds(base + c * CHUNK, CHUNK)], m_buf)

        def body(i, _):
            for u in range(UNROLL):
                off = (i * UNROLL + u) * LANES
                _, bits = _neg_bits(g_buf, m_buf, off)
                bkt = (bits >> 16).astype(jnp.int32)
                plsc.addupdate_scatter(hist, [bkt], ones)
            return 0

        lax.fori_loop(0, CHUNK // LANES // UNROLL, body, 0)
    pltpu.sync_copy(hist, out_hbm.at[wid])


def _sc_hist_lo_body(g_hbm, m_hbm, b1_hbm, out_hbm, g_buf, m_buf, b1_buf,
                     hist):
    wid = _worker_id()
    per_w = g_hbm.shape[0] // NW
    nchunk = per_w // CHUNK
    _zero_hist(hist)
    pltpu.sync_copy(b1_hbm, b1_buf)
    b1v = b1_buf[...]
    ones = jnp.ones((LANES,), jnp.int32)
    base = wid * per_w
    for c in range(nchunk):
        pltpu.sync_copy(g_hbm.at[pl.ds(base + c * CHUNK, CHUNK)], g_buf)
        pltpu.sync_copy(m_hbm.at[pl.ds(base + c * CHUNK, CHUNK)], m_buf)

        def body(i, _):
            for u in range(UNROLL):
                off = (i * UNROLL + u) * LANES
                _, bits = _neg_bits(g_buf, m_buf, off)
                mk = (bits >> 16) == b1v
                bkt = (bits & jnp.uint32(0xFFFF)).astype(jnp.int32)
                plsc.addupdate_scatter(hist, [bkt], ones, mask=mk)
            return 0

        lax.fori_loop(0, CHUNK // LANES // UNROLL, body, 0)
    pltpu.sync_copy(hist, out_hbm.at[wid])


def _sc_sum_gt_body(g_hbm, m_hbm, t_hbm, s_out, c_out, g_buf, m_buf, t_buf,
                    s_buf, c_buf):
    wid = _worker_id()
    per_w = g_hbm.shape[0] // NW
    nchunk = per_w // CHUNK
    pltpu.sync_copy(t_hbm, t_buf)
    tv = t_buf[...]
    base = wid * per_w
    s_acc = jnp.zeros((LANES,), jnp.float32)
    c_acc = jnp.zeros((LANES,), jnp.int32)
    one = jnp.ones((LANES,), jnp.int32)
    zero = jnp.zeros((LANES,), jnp.int32)
    zf = jnp.zeros((LANES,), jnp.float32)
    for c in range(nchunk):
        pltpu.sync_copy(g_hbm.at[pl.ds(base + c * CHUNK, CHUNK)], g_buf)
        pltpu.sync_copy(m_hbm.at[pl.ds(base + c * CHUNK, CHUNK)], m_buf)

        def body(i, carry):
            sa, ca = carry
            for u in range(UNROLL):
                off = (i * UNROLL + u) * LANES
                neg, bits = _neg_bits(g_buf, m_buf, off)
                mk = bits > tv
                sa = sa + jnp.where(mk, neg, zf)
                ca = ca + jnp.where(mk, one, zero)
            return sa, ca

        s_acc, c_acc = lax.fori_loop(0, CHUNK // LANES // UNROLL, body,
                                     (s_acc, c_acc))
    s_buf[...] = s_acc
    c_buf[...] = c_acc
    pltpu.sync_copy(s_buf, s_out.at[wid])
    pltpu.sync_copy(c_buf, c_out.at[wid])


def _sc_hist_hi(gf, mf):
    return pl.kernel(
        _sc_hist_hi_body,
        out_type=jax.ShapeDtypeStruct((NW, NBINS), jnp.int32),
        mesh=_sc_mesh(),
        compiler_params=pltpu.CompilerParams(needs_layout_passes=False),
        scratch_types=[
            pltpu.VMEM((CHUNK,), jnp.float32),
            pltpu.VMEM((CHUNK,), jnp.float32),
            pltpu.VMEM((NBINS,), jnp.int32),
        ],
    )(gf, mf)


def _sc_hist_lo(gf, mf, b1):
    return pl.kernel(
        _sc_hist_lo_body,
        out_type=jax.ShapeDtypeStruct((NW, NBINS), jnp.int32),
        mesh=_sc_mesh(),
        compiler_params=pltpu.CompilerParams(needs_layout_passes=False),
        scratch_types=[
            pltpu.VMEM((CHUNK,), jnp.float32),
            pltpu.VMEM((CHUNK,), jnp.float32),
            pltpu.VMEM((LANES,), jnp.uint32),
            pltpu.VMEM((NBINS,), jnp.int32),
        ],
    )(gf, mf, b1)


def _sc_sum_gt(gf, mf, t_bits):
    return pl.kernel(
        _sc_sum_gt_body,
        out_type=(
            jax.ShapeDtypeStruct((NW, LANES), jnp.float32),
            jax.ShapeDtypeStruct((NW, LANES), jnp.int32),
        ),
        mesh=_sc_mesh(),
        compiler_params=pltpu.CompilerParams(needs_layout_passes=False),
        scratch_types=[
            pltpu.VMEM((CHUNK,), jnp.float32),
            pltpu.VMEM((CHUNK,), jnp.float32),
            pltpu.VMEM((LANES,), jnp.uint32),
            pltpu.VMEM((LANES,), jnp.float32),
            pltpu.VMEM((LANES,), jnp.int32),
        ],
    )(gf, mf, t_bits)


def _select_bucket(cnt, k):
    s = jnp.cumsum(cnt[::-1])[::-1]
    iot = jnp.arange(cnt.shape[0], dtype=jnp.int32)
    b = jnp.max(jnp.where(s >= k, iot, -1))
    c_above = s[b] - cnt[b]
    return b, c_above


def kernel(preds, label_threshold_map, label_threshold_mask,
           label_shrink_map, label_shrink_mask):
    sums = _dense_sums(preds, label_threshold_map, label_threshold_mask,
                       label_shrink_map, label_shrink_mask)
    i1, p1s, g1, m1, l1, mt, i2, p2s = [sums[i] for i in range(8)]

    loss = 1.0 - 2.0 * i1 / (p1s + g1 + EPS)
    pos_count = jnp.floor(g1)
    sum_neg = m1 - g1
    neg_count = jnp.floor(jnp.minimum(sum_neg, pos_count * OHEM_RATIO))
    k = neg_count.astype(jnp.int32)
    k_eff = jnp.maximum(k, 1)

    gf = label_shrink_map.reshape(-1)
    mf = label_shrink_mask.reshape(-1)

    hist1 = _sc_hist_hi(gf, mf)
    cnt1 = jnp.sum(hist1, axis=0)
    b1, c_above1 = _select_bucket(cnt1, k_eff)
    k2 = k_eff - c_above1

    b1u = jnp.broadcast_to(b1.astype(jnp.uint32), (LANES,))
    hist2 = _sc_hist_lo(gf, mf, b1u)
    cnt2 = jnp.sum(hist2, axis=0)
    b2, _ = _select_bucket(cnt2, k2)

    t_bits = (b1.astype(jnp.uint32) << 16) | b2.astype(jnp.uint32)
    t = lax.bitcast_convert_type(t_bits, jnp.float32)
    t_vec = jnp.broadcast_to(t_bits, (LANES,))

    s_parts, c_parts = _sc_sum_gt(gf, mf, t_vec)
    s_gt = jnp.sum(s_parts)
    c_gt = jnp.sum(c_parts)

    topk = s_gt + t * (neg_count - c_gt.astype(jnp.float32))
    neg_sum = loss * topk
    pos_loss_sum = loss * g1
    balance_pos_neg = (pos_loss_sum + neg_sum) / (pos_count + neg_count + EPS)
    balance_pos_only = pos_loss_sum / (pos_count + EPS)
    balance = jnp.where(neg_count > 0, balance_pos_neg, balance_pos_only)

    loss_shrink = ALPHA * balance
    loss_threshold = BETA * l1 / (mt + EPS)
    loss_binary = 1.0 - 2.0 * i2 / (p2s + g1 + EPS)
    cbn = jnp.array(0.0, dtype=jnp.float32)
    loss_all = loss_shrink + loss_threshold + loss_binary + cbn
    return (loss_all, loss_shrink, loss_threshold, loss_binary, cbn)


# speedup vs baseline: 26.4126x; 1.5355x over previous
import jax
import jax.numpy as jnp
from jax import lax
from jax.experimental import pallas as pl
from jax.experimental.pallas import tpu as pltpu
from jax.experimental.pallas import tpu_sc as plsc

EPS = 1e-06
ALPHA = 5.0
BETA = 10.0
OHEM_RATIO = 3.0

NC, NS, LANES = 2, 16, 16
NW = NC * NS
CHUNK = 8192
UNROLL = 8
NBINS = 65536


def _dense_body(pr_ref, tm_ref, tmask_ref, g_ref, m_ref, out_ref):
    p0 = pr_ref[0]
    p1 = pr_ref[1]
    p2 = pr_ref[2]
    tm = tm_ref[0]
    tmask = tmask_ref[0]
    g = g_ref[0]
    m = m_ref[0]
    gm = g * m
    p0m = p0 * m
    p2m = p2 * m
    vals = [
        jnp.sum(p0m * g),
        jnp.sum(p0m),
        jnp.sum(gm),
        jnp.sum(m),
        jnp.sum(jnp.abs(p1 - tm) * tmask),
        jnp.sum(tmask),
        jnp.sum(p2m * g),
        jnp.sum(p2m),
    ]
    rows = lax.broadcasted_iota(jnp.int32, (8, 128), 0)
    total = jnp.zeros((8, 128), jnp.float32)
    for i, v in enumerate(vals):
        total = total + jnp.where(rows == i, v, 0.0)

    @pl.when(pl.program_id(0) == 0)
    def _():
        out_ref[...] = jnp.zeros_like(out_ref)

    out_ref[...] += total


def _dense_sums(preds, tm, tmask, g, m):
    B, C, H, W = preds.shape
    pr = preds.reshape(B * C, H, W)
    out = pl.pallas_call(
        _dense_body,
        grid=(B,),
        in_specs=[
            pl.BlockSpec((C, H, W), lambda i: (i, 0, 0)),
            pl.BlockSpec((1, H, W), lambda i: (i, 0, 0)),
            pl.BlockSpec((1, H, W), lambda i: (i, 0, 0)),
            pl.BlockSpec((1, H, W), lambda i: (i, 0, 0)),
            pl.BlockSpec((1, H, W), lambda i: (i, 0, 0)),
        ],
        out_specs=pl.BlockSpec((8, 128), lambda i: (0, 0)),
        out_shape=jax.ShapeDtypeStruct((8, 128), jnp.float32),
    )(pr, tm, tmask, g, m)
    return out[:, 0]


import functools


@functools.lru_cache(maxsize=None)
def _sc_mesh():
    return plsc.VectorSubcoreMesh(core_axis_name="c", subcore_axis_name="s",
                                  num_cores=NC, num_subcores=NS)


def _worker_id():
    return lax.axis_index("s") * NC + lax.axis_index("c")


def _zero_hist(hist):
    zeros = jnp.zeros((LANES,), jnp.int32)

    def zbody(i, _):
        for u in range(UNROLL):
            hist[pl.ds((i * UNROLL + u) * LANES, LANES)] = zeros
        return 0

    lax.fori_loop(0, NBINS // LANES // UNROLL, zbody, 0)


def _neg_bits(g_buf, m_buf, off):
    gv = g_buf[pl.ds(off, LANES)]
    mv = m_buf[pl.ds(off, LANES)]
    neg = mv - gv * mv
    return neg, plsc.bitcast(neg, jnp.uint32)


def _sc_hist_hi_body(g_hbm, m_hbm, out_hbm, g_buf, m_buf, hist):
    wid = _worker_id()
    per_w = g_hbm.shape[0] // NW
    nchunk = per_w // CHUNK
    _zero_hist(hist)
    ones = jnp.ones((LANES,), jnp.int32)
    base = wid * per_w
    for c in range(nchunk):
        pltpu.sync_copy(g_hbm.at[pl.ds(base + c * CHUNK, CHUNK)], g_buf)
        pltpu.sync_copy(m_hbm.at[pl.ds(base + c * CHUNK, CHUNK)], m_buf)

        @plsc.parallel_loop(0, CHUNK // LANES, 1, unroll=UNROLL)
        def _(v):
            _, bits = _neg_bits(g_buf, m_buf, v * LANES)
            bkt = (bits >> 16).astype(jnp.int32)
            plsc.addupdate_scatter(hist, [bkt], ones)

    pltpu.sync_copy(hist, out_hbm.at[wid])


def _sc_hist_lo_body(g_hbm, m_hbm, b1_hbm, out_hbm, g_buf, m_buf, b1_buf,
                     hist):
    wid = _worker_id()
    per_w = g_hbm.shape[0] // NW
    nchunk = per_w // CHUNK
    _zero_hist(hist)
    pltpu.sync_copy(b1_hbm, b1_buf)
    b1v = b1_buf[...]
    ones = jnp.ones((LANES,), jnp.int32)
    base = wid * per_w
    for c in range(nchunk):
        pltpu.sync_copy(g_hbm.at[pl.ds(base + c * CHUNK, CHUNK)], g_buf)
        pltpu.sync_copy(m_hbm.at[pl.ds(base + c * CHUNK, CHUNK)], m_buf)

        @plsc.parallel_loop(0, CHUNK // LANES, 1, unroll=UNROLL)
        def _(v):
            _, bits = _neg_bits(g_buf, m_buf, v * LANES)
            mk = (bits >> 16) == b1v
            bkt = (bits & jnp.uint32(0xFFFF)).astype(jnp.int32)
            plsc.addupdate_scatter(hist, [bkt], ones, mask=mk)

    pltpu.sync_copy(hist, out_hbm.at[wid])


def _sc_sum_gt_body(g_hbm, m_hbm, t_hbm, s_out, c_out, g_buf, m_buf, t_buf,
                    s_buf, c_buf):
    wid = _worker_id()
    per_w = g_hbm.shape[0] // NW
    nchunk = per_w // CHUNK
    pltpu.sync_copy(t_hbm, t_buf)
    tv = t_buf[...]
    base = wid * per_w
    s_acc = jnp.zeros((LANES,), jnp.float32)
    c_acc = jnp.zeros((LANES,), jnp.int32)
    one = jnp.ones((LANES,), jnp.int32)
    zero = jnp.zeros((LANES,), jnp.int32)
    zf = jnp.zeros((LANES,), jnp.float32)
    for c in range(nchunk):
        pltpu.sync_copy(g_hbm.at[pl.ds(base + c * CHUNK, CHUNK)], g_buf)
        pltpu.sync_copy(m_hbm.at[pl.ds(base + c * CHUNK, CHUNK)], m_buf)

        def body(v, carry):
            sa, ca = carry
            neg, bits = _neg_bits(g_buf, m_buf, v * LANES)
            mk = bits > tv
            sa = sa + jnp.where(mk, neg, zf)
            ca = ca + jnp.where(mk, one, zero)
            return sa, ca

        s_acc, c_acc = plsc.parallel_loop(
            0, CHUNK // LANES, 1, unroll=UNROLL,
            carry=(s_acc, c_acc))(body)
    s_buf[...] = s_acc
    c_buf[...] = c_acc
    pltpu.sync_copy(s_buf, s_out.at[wid])
    pltpu.sync_copy(c_buf, c_out.at[wid])


def _sc_hist_hi(gf, mf):
    return pl.kernel(
        _sc_hist_hi_body,
        out_type=jax.ShapeDtypeStruct((NW, NBINS), jnp.int32),
        mesh=_sc_mesh(),
        compiler_params=pltpu.CompilerParams(needs_layout_passes=False),
        scratch_types=[
            pltpu.VMEM((CHUNK,), jnp.float32),
            pltpu.VMEM((CHUNK,), jnp.float32),
            pltpu.VMEM((NBINS,), jnp.int32),
        ],
    )(gf, mf)


def _sc_hist_lo(gf, mf, b1):
    return pl.kernel(
        _sc_hist_lo_body,
        out_type=jax.ShapeDtypeStruct((NW, NBINS), jnp.int32),
        mesh=_sc_mesh(),
        compiler_params=pltpu.CompilerParams(needs_layout_passes=False),
        scratch_types=[
            pltpu.VMEM((CHUNK,), jnp.float32),
            pltpu.VMEM((CHUNK,), jnp.float32),
            pltpu.VMEM((LANES,), jnp.uint32),
            pltpu.VMEM((NBINS,), jnp.int32),
        ],
    )(gf, mf, b1)


def _sc_sum_gt(gf, mf, t_bits):
    return pl.kernel(
        _sc_sum_gt_body,
        out_type=(
            jax.ShapeDtypeStruct((NW, LANES), jnp.float32),
            jax.ShapeDtypeStruct((NW, LANES), jnp.int32),
        ),
        mesh=_sc_mesh(),
        compiler_params=pltpu.CompilerParams(needs_layout_passes=False),
        scratch_types=[
            pltpu.VMEM((CHUNK,), jnp.float32),
            pltpu.VMEM((CHUNK,), jnp.float32),
            pltpu.VMEM((LANES,), jnp.uint32),
            pltpu.VMEM((LANES,), jnp.float32),
            pltpu.VMEM((LANES,), jnp.int32),
        ],
    )(gf, mf, t_bits)


def _select_bucket(cnt, k):
    s = jnp.cumsum(cnt[::-1])[::-1]
    iot = jnp.arange(cnt.shape[0], dtype=jnp.int32)
    b = jnp.max(jnp.where(s >= k, iot, -1))
    c_above = s[b] - cnt[b]
    return b, c_above


def kernel(preds, label_threshold_map, label_threshold_mask,
           label_shrink_map, label_shrink_mask):
    sums = _dense_sums(preds, label_threshold_map, label_threshold_mask,
                       label_shrink_map, label_shrink_mask)
    i1, p1s, g1, m1, l1, mt, i2, p2s = [sums[i] for i in range(8)]

    loss = 1.0 - 2.0 * i1 / (p1s + g1 + EPS)
    pos_count = jnp.floor(g1)
    sum_neg = m1 - g1
    neg_count = jnp.floor(jnp.minimum(sum_neg, pos_count * OHEM_RATIO))
    k = neg_count.astype(jnp.int32)
    k_eff = jnp.maximum(k, 1)

    gf = label_shrink_map.reshape(-1)
    mf = label_shrink_mask.reshape(-1)

    hist1 = _sc_hist_hi(gf, mf)
    cnt1 = jnp.sum(hist1, axis=0)
    b1, c_above1 = _select_bucket(cnt1, k_eff)
    k2 = k_eff - c_above1

    b1u = jnp.broadcast_to(b1.astype(jnp.uint32), (LANES,))
    hist2 = _sc_hist_lo(gf, mf, b1u)
    cnt2 = jnp.sum(hist2, axis=0)
    b2, _ = _select_bucket(cnt2, k2)

    t_bits = (b1.astype(jnp.uint32) << 16) | b2.astype(jnp.uint32)
    t = lax.bitcast_convert_type(t_bits, jnp.float32)
    t_vec = jnp.broadcast_to(t_bits, (LANES,))

    s_parts, c_parts = _sc_sum_gt(gf, mf, t_vec)
    s_gt = jnp.sum(s_parts)
    c_gt = jnp.sum(c_parts)

    topk = s_gt + t * (neg_count - c_gt.astype(jnp.float32))
    neg_sum = loss * topk
    pos_loss_sum = loss * g1
    balance_pos_neg = (pos_loss_sum + neg_sum) / (pos_count + neg_count + EPS)
    balance_pos_only = pos_loss_sum / (pos_count + EPS)
    balance = jnp.where(neg_count > 0, balance_pos_neg, balance_pos_only)

    loss_shrink = ALPHA * balance
    loss_threshold = BETA * l1 / (mt + EPS)
    loss_binary = 1.0 - 2.0 * i2 / (p2s + g1 + EPS)
    cbn = jnp.array(0.0, dtype=jnp.float32)
    loss_all = loss_shrink + loss_threshold + loss_binary + cbn
    return (loss_all, loss_shrink, loss_threshold, loss_binary, cbn)
